# baseline (speedup 1.0000x reference)
import functools

import jax
import jax.numpy as jnp
from jax import lax
from jax.experimental import pallas as pl
from jax.experimental.pallas import tpu as pltpu
from jax.experimental.pallas import tpu_sc as plsc

_HIDDEN = 2048
_TOTAL = 16384
_NW = 32
_B_PER_W = _TOTAL // _NW
_CHUNK = 16
_HALF = _CHUNK // 2
_NBUF = 3
_N_CHUNKS = _B_PER_W // _CHUNK

_mesh = plsc.VectorSubcoreMesh(core_axis_name="c", subcore_axis_name="s")


@functools.partial(
    pl.kernel,
    mesh=_mesh,
    out_type=jax.ShapeDtypeStruct((4, 4096, _HIDDEN), jnp.float32),
    scratch_types=[
        pltpu.VMEM((_B_PER_W,), jnp.int32),
        pltpu.VMEM((_NBUF, _CHUNK, _HIDDEN), jnp.float32),
        pltpu.SemaphoreType.DMA,
        pltpu.SemaphoreType.DMA,
        pltpu.SemaphoreType.DMA,
        pltpu.SemaphoreType.DMA,
        pltpu.SemaphoreType.DMA,
        pltpu.SemaphoreType.DMA,
    ],
)
def _emb_lookup(tokens_hbm, lut_hbm, out_hbm, idx_v, rows_v,
                g0, g1, g2, w0, w1, w2):
    wid = lax.axis_index("s") * 2 + lax.axis_index("c")
    row = wid // 8
    col = (wid % 8) * _B_PER_W
    pltpu.sync_copy(tokens_hbm.at[row, pl.ds(col, _B_PER_W)], idx_v)

    gsems = [g0, g1, g2]
    wsems = [w0, w1, w2]

    def gather_half(c, b, h):
        return pltpu.make_async_copy(
            lut_hbm.at[idx_v.at[pl.ds(c * _CHUNK + h * _HALF, _HALF)]],
            rows_v.at[b, pl.ds(h * _HALF, _HALF)],
            gsems[b],
        )

    def gather_start(c, b):
        gather_half(c, b, 0).start()
        gather_half(c, b, 1).start()

    def gather_wait(c, b):
        gather_half(c, b, 0).wait()
        gather_half(c, b, 1).wait()

    def writeback(c, b):
        return pltpu.make_async_copy(
            rows_v.at[b],
            out_hbm.at[row, pl.ds(col + c * _CHUNK, _CHUNK)],
            wsems[b],
        )

    for b in range(_NBUF):
        gather_start(b, b)

    def step(c, b):
        gather_wait(c, b)
        writeback(c, b).start()
        nc = c + _NBUF
        writeback(c, b).wait()

        if isinstance(nc, int):
            if nc < _N_CHUNKS:
                gather_start(nc, b)
        else:
            @pl.when(nc < _N_CHUNKS)
            def _():
                gather_start(nc, b)

    main = (_N_CHUNKS // _NBUF) * _NBUF

    def body(i, _):
        for b in range(_NBUF):
            step(i * _NBUF + b, b)
        return 0

    lax.fori_loop(0, main // _NBUF, body, 0)
    for c in range(main, _N_CHUNKS):
        step(c, c % _NBUF)


def kernel(tokens, lut):
    return _emb_lookup(tokens.astype(jnp.int32), lut)


# speedup vs baseline: 1.0002x; 1.0002x over previous
import functools

import jax
import jax.numpy as jnp
from jax import lax
from jax.experimental import pallas as pl
from jax.experimental.pallas import tpu as pltpu
from jax.experimental.pallas import tpu_sc as plsc

_HIDDEN = 2048
_TOTAL = 16384
_NW = 32
_B_PER_W = _TOTAL // _NW
_CHUNK = 8
_NBUF = 4
_N_CHUNKS = _B_PER_W // _CHUNK

_mesh = plsc.VectorSubcoreMesh(core_axis_name="c", subcore_axis_name="s")


@functools.partial(
    pl.kernel,
    mesh=_mesh,
    out_type=jax.ShapeDtypeStruct((4, 4096, _HIDDEN), jnp.float32),
    scratch_types=[
        pltpu.VMEM((_B_PER_W,), jnp.int32),
        pltpu.VMEM((_NBUF, _CHUNK, _HIDDEN), jnp.float32),
        pltpu.SemaphoreType.DMA,
        pltpu.SemaphoreType.DMA,
        pltpu.SemaphoreType.DMA,
        pltpu.SemaphoreType.DMA,
        pltpu.SemaphoreType.DMA,
        pltpu.SemaphoreType.DMA,
        pltpu.SemaphoreType.DMA,
        pltpu.SemaphoreType.DMA,
    ],
)
def _emb_lookup(tokens_hbm, lut_hbm, out_hbm, idx_v, rows_v,
                g0, g1, g2, g3, w0, w1, w2, w3):
    wid = lax.axis_index("s") * 2 + lax.axis_index("c")
    row = wid // 8
    col = (wid % 8) * _B_PER_W
    pltpu.sync_copy(tokens_hbm.at[row, pl.ds(col, _B_PER_W)], idx_v)

    gsems = [g0, g1, g2, g3]
    wsems = [w0, w1, w2, w3]

    def gather(c, b):
        return pltpu.make_async_copy(
            lut_hbm.at[idx_v.at[pl.ds(c * _CHUNK, _CHUNK)]],
            rows_v.at[b],
            gsems[b],
        )

    def writeback(c, b):
        return pltpu.make_async_copy(
            rows_v.at[b],
            out_hbm.at[row, pl.ds(col + c * _CHUNK, _CHUNK)],
            wsems[b],
        )

    for b in range(_NBUF):
        gather(b, b).start()

    def step(c, b):
        gather(c, b).wait()
        writeback(c, b).start()
        nc = c + _NBUF
        writeback(c, b).wait()

        if isinstance(nc, int):
            if nc < _N_CHUNKS:
                gather(nc, b).start()
        else:
            @pl.when(nc < _N_CHUNKS)
            def _():
                gather(nc, b).start()

    def body(i, _):
        for b in range(_NBUF):
            step(i * _NBUF + b, b)
        return 0

    lax.fori_loop(0, _N_CHUNKS // _NBUF, body, 0)


def kernel(tokens, lut):
    return _emb_lookup(tokens.astype(jnp.int32), lut)


# speedup vs baseline: 1.0108x; 1.0106x over previous
import functools

import jax
import jax.numpy as jnp
from jax import lax
from jax.experimental import pallas as pl
from jax.experimental.pallas import tpu as pltpu
from jax.experimental.pallas import tpu_sc as plsc

_HIDDEN = 2048
_TOTAL = 16384
_NW = 32
_B_PER_W = _TOTAL // _NW
_CHUNK = 8
_NBUF = 4
_N_CHUNKS = _B_PER_W // _CHUNK
_N_GROUPS = _N_CHUNKS // _NBUF

_mesh = plsc.VectorSubcoreMesh(core_axis_name="c", subcore_axis_name="s")


@functools.partial(
    pl.kernel,
    mesh=_mesh,
    out_type=jax.ShapeDtypeStruct((4, 4096, _HIDDEN), jnp.float32),
    scratch_types=[
        pltpu.VMEM((_B_PER_W,), jnp.int32),
        pltpu.VMEM((_NBUF, _CHUNK, _HIDDEN), jnp.float32),
        pltpu.VMEM_SHARED((16, _CHUNK, _HIDDEN), jnp.float32),
        pltpu.SemaphoreType.DMA,
        pltpu.SemaphoreType.DMA,
        pltpu.SemaphoreType.DMA,
        pltpu.SemaphoreType.DMA,
        pltpu.SemaphoreType.DMA,
        pltpu.SemaphoreType.DMA,
        pltpu.SemaphoreType.DMA,
    ],
)
def _emb_lookup(tokens_hbm, lut_hbm, out_hbm, idx_v, rows_v, spm,
                g0, g1, g2, g3, w0, w1, dsem):
    wid = lax.axis_index("s") * 2 + lax.axis_index("c")
    tid = lax.axis_index("s")
    row = wid // 8
    col = (wid % 8) * _B_PER_W
    pltpu.sync_copy(tokens_hbm.at[row, pl.ds(col, _B_PER_W)], idx_v)

    gsems = [g0, g1, g2, g3]
    wsems = [w0, w1]

    def gather(c, b):
        return pltpu.make_async_copy(
            lut_hbm.at[idx_v.at[pl.ds(c * _CHUNK, _CHUNK)]],
            rows_v.at[b],
            gsems[b],
        )

    def stream_wb(c, b, sem):
        return pltpu.make_async_copy(
            rows_v.at[b],
            out_hbm.at[row, pl.ds(col + c * _CHUNK, _CHUNK)],
            sem,
        )

    def dma_wb(c):
        return pltpu.make_async_copy(
            spm.at[tid],
            out_hbm.at[row, pl.ds(col + c * _CHUNK, _CHUNK)],
            dsem,
        )

    for b in range(_NBUF):
        gather(b, b).start()

    def body(i, _):
        for k in range(_NBUF):
            c = i * _NBUF + k
            b = k
            gather(c, b).wait()
            if k % 2 == 0:
                sem = wsems[k // 2]
                stream_wb(c, b, sem).start()
                stream_wb(c, b, sem).wait()
            else:
                if k == 1:
                    @pl.when(i > 0)
                    def _():
                        dma_wb(c - 2).wait()
                else:
                    dma_wb(c - 2).wait()
                pltpu.sync_copy(rows_v.at[b], spm.at[tid])
                dma_wb(c).start()

            @pl.when(i < _N_GROUPS - 1)
            def _():
                gather(c + _NBUF, b).start()

        return 0

    lax.fori_loop(0, _N_GROUPS, body, 0)
    dma_wb(_N_CHUNKS - 1).wait()


def kernel(tokens, lut):
    return _emb_lookup(tokens.astype(jnp.int32), lut)
